# baseline (speedup 1.0000x reference)
import jax
import jax.numpy as jnp
from jax.experimental import pallas as pl
from jax.experimental.pallas import tpu as pltpu

_SAMPLE_N = 16384
_FEAT = 128
_NCHUNK = 8
_CHUNK = _SAMPLE_N // _NCHUNK


def _body(x_hbm, o_hbm, buf, in_sems, out_sems):
    def in_copy(i, slot):
        return pltpu.make_async_copy(
            x_hbm.at[pl.ds(i * _CHUNK, _CHUNK), :],
            buf.at[slot],
            in_sems.at[slot],
        )

    def out_copy(i, slot):
        return pltpu.make_async_copy(
            buf.at[slot],
            o_hbm.at[pl.ds(i * _CHUNK, _CHUNK), :],
            out_sems.at[slot],
        )

    in_copy(0, 0).start()
    for i in range(_NCHUNK):
        slot = i % 2
        nxt = (i + 1) % 2
        if i + 1 < _NCHUNK:
            if i >= 1:
                out_copy(i - 1, nxt).wait()
            in_copy(i + 1, nxt).start()
        in_copy(i, slot).wait()
        out_copy(i, slot).start()
    out_copy(_NCHUNK - 2, (_NCHUNK - 2) % 2).wait()
    out_copy(_NCHUNK - 1, (_NCHUNK - 1) % 2).wait()


def kernel(dataset):
    return pl.pallas_call(
        _body,
        in_specs=[pl.BlockSpec(memory_space=pltpu.MemorySpace.HBM)],
        out_specs=pl.BlockSpec(memory_space=pltpu.MemorySpace.HBM),
        out_shape=jax.ShapeDtypeStruct((_SAMPLE_N, _FEAT), jnp.float32),
        scratch_shapes=[
            pltpu.VMEM((2, _CHUNK, _FEAT), jnp.float32),
            pltpu.SemaphoreType.DMA((2,)),
            pltpu.SemaphoreType.DMA((2,)),
        ],
    )(dataset)


# speedup vs baseline: 1.7503x; 1.7503x over previous
import jax
import jax.numpy as jnp
from jax.experimental import pallas as pl
from jax.experimental.pallas import tpu as pltpu

_SAMPLE_N = 16384
_FEAT = 128
_NCHUNK = 8
_CHUNK = _SAMPLE_N // _NCHUNK


def _body(x_hbm, o_hbm, buf, in_sems, out_sems):
    def in_copy(i, slot):
        return pltpu.make_async_copy(
            x_hbm.at[pl.ds(i * _CHUNK, _CHUNK), :],
            buf.at[slot],
            in_sems.at[slot],
        )

    def out_copy(i, slot):
        return pltpu.make_async_copy(
            buf.at[slot],
            o_hbm.at[pl.ds(i * _CHUNK, _CHUNK), :],
            out_sems.at[slot],
        )

    for i in range(_NCHUNK):
        in_copy(i, i).start()
    for i in range(_NCHUNK):
        in_copy(i, i).wait()
        out_copy(i, i).start()
    for i in range(_NCHUNK):
        out_copy(i, i).wait()


def kernel(dataset):
    return pl.pallas_call(
        _body,
        in_specs=[pl.BlockSpec(memory_space=pltpu.MemorySpace.HBM)],
        out_specs=pl.BlockSpec(memory_space=pltpu.MemorySpace.HBM),
        out_shape=jax.ShapeDtypeStruct((_SAMPLE_N, _FEAT), jnp.float32),
        scratch_shapes=[
            pltpu.VMEM((_NCHUNK, _CHUNK, _FEAT), jnp.float32),
            pltpu.SemaphoreType.DMA((_NCHUNK,)),
            pltpu.SemaphoreType.DMA((_NCHUNK,)),
        ],
    )(dataset)
